# Initial kernel scaffold; baseline (speedup 1.0000x reference)
import jax
import jax.numpy as jnp
from jax.experimental import pallas as pl


def kernel(bboxes, region_table, entity_table, rW, rb, r_gamma, r_beta, eW, eb, e_gamma, e_beta, region_ids, entity_ids):
    raise NotImplementedError("write your pallas kernel here")


# speedup vs baseline: 2.6159x; 2.6159x over previous
import functools

import jax
import jax.numpy as jnp
from jax import lax
from jax.experimental import pallas as pl
from jax.experimental.pallas import tpu as pltpu
from jax.experimental.pallas import tpu_sc as plsc

_SQRT_HALF = 0.7071067811865476


def _table_proc_body(rt, rwt, rb2, rg2, rbe2, et, ewt, eb2, eg2, ebe2,
                     ro, eo):
    def proc(t_ref, wt_ref, b_ref, g_ref, be_ref, o_ref):
        z = jnp.dot(t_ref[...], wt_ref[...],
                    preferred_element_type=jnp.float32) + b_ref[...]
        m = jnp.mean(z, axis=-1, keepdims=True)
        zc = z - m
        v = jnp.mean(zc * zc, axis=-1, keepdims=True)
        y = zc / jnp.sqrt(v + 1e-5) * g_ref[...] + be_ref[...]
        o_ref[...] = y * 0.5 * (1.0 + lax.erf(y * _SQRT_HALF))

    proc(rt, rwt, rb2, rg2, rbe2, ro)
    proc(et, ewt, eb2, eg2, ebe2, eo)


def _process_tables(region_table, entity_table, rW, rb, r_gamma, r_beta,
                    eW, eb, e_gamma, e_beta):
    VR, D = region_table.shape
    VE, _ = entity_table.shape
    f32 = jnp.float32
    return pl.pallas_call(
        _table_proc_body,
        out_shape=[jax.ShapeDtypeStruct((VR, D), f32),
                   jax.ShapeDtypeStruct((VE, D), f32)],
    )(region_table, rW.T, rb.reshape(1, D), r_gamma.reshape(1, D),
      r_beta.reshape(1, D), entity_table, eW.T, eb.reshape(1, D),
      e_gamma.reshape(1, D), e_beta.reshape(1, D))


def _make_sc_gather(T, D, VR, VE, F):
    NC, NS, L = 2, 16, 16
    NW = NC * NS
    TPW = T // NW
    C = 320
    NCHUNK = TPW // C
    G = C // L
    assert TPW * NW == T and NCHUNK * C == TPW and G * L == C

    mesh = plsc.VectorSubcoreMesh(core_axis_name="c", subcore_axis_name="s")

    @functools.partial(
        pl.kernel, mesh=mesh,
        compiler_params=pltpu.CompilerParams(needs_layout_passes=False),
        out_type=jax.ShapeDtypeStruct((T * F,), jnp.float32),
        scratch_types=[
            pltpu.VMEM((D * VR,), jnp.float32),
            pltpu.VMEM((D * VE,), jnp.float32),
            pltpu.VMEM((C,), jnp.int32),
            pltpu.VMEM((C,), jnp.int32),
            pltpu.VMEM((C * 4,), jnp.float32),
            pltpu.VMEM((C * F,), jnp.float32),
        ],
    )
    def sc_body(prc, pec, rid_hbm, eid_hbm, bb_hbm, out_hbm,
                tbl_r, tbl_e, rid_v, eid_v, bb_v, stg):
        wid = lax.axis_index("s") * NC + lax.axis_index("c")
        base = wid * TPW
        pltpu.sync_copy(prc, tbl_r)
        pltpu.sync_copy(pec, tbl_e)
        iota = lax.iota(jnp.int32, L)

        def group_body(g, _):
            lt0 = g * L
            rids = rid_v[pl.ds(lt0, L)]
            eids = eid_v[pl.ds(lt0, L)]
            ob = (lt0 * F + 6) + iota * F
            for d in range(D):
                v = plsc.load_gather(tbl_r, [rids + d * VR])
                plsc.store_scatter(stg, [ob + d], v)
            for d in range(D):
                v = plsc.load_gather(tbl_e, [eids + d * VE])
                plsc.store_scatter(stg, [ob + (D + d)], v)
            bb = lt0 * 4 + iota * 4
            x1 = plsc.load_gather(bb_v, [bb])
            y1 = plsc.load_gather(bb_v, [bb + 1])
            x2 = plsc.load_gather(bb_v, [bb + 2])
            y2 = plsc.load_gather(bb_v, [bb + 3])
            w = x2 - x1
            h = y2 - y1
            fb = ob - 6
            plsc.store_scatter(stg, [fb], x1)
            plsc.store_scatter(stg, [fb + 1], y1)
            plsc.store_scatter(stg, [fb + 2], x2)
            plsc.store_scatter(stg, [fb + 3], y2)
            plsc.store_scatter(stg, [fb + 4], w * h)
            plsc.store_scatter(stg, [fb + 5], w / (h + 1e-6))
            return 0

        def chunk_body(k, _):
            tok0 = base + k * C
            pltpu.sync_copy(rid_hbm.at[pl.ds(tok0, C)], rid_v)
            pltpu.sync_copy(eid_hbm.at[pl.ds(tok0, C)], eid_v)
            pltpu.sync_copy(bb_hbm.at[pl.ds(tok0 * 4, C * 4)], bb_v)
            lax.fori_loop(0, G, group_body, 0)
            pltpu.sync_copy(stg, out_hbm.at[pl.ds(tok0 * F, C * F)])
            return 0

        lax.fori_loop(0, NCHUNK, chunk_body, 0)

    return sc_body


def kernel(bboxes, region_table, entity_table, rW, rb, r_gamma, r_beta,
           eW, eb, e_gamma, e_beta, region_ids, entity_ids):
    B, N, _ = bboxes.shape
    VR, D = region_table.shape
    VE, _ = entity_table.shape
    T = B * N
    F = 2 * D + 6

    pr, pe = _process_tables(region_table, entity_table, rW, rb, r_gamma,
                             r_beta, eW, eb, e_gamma, e_beta)
    prc = pr.T.reshape(D * VR)
    pec = pe.T.reshape(D * VE)

    sc = _make_sc_gather(T, D, VR, VE, F)
    out = sc(prc, pec,
             region_ids.astype(jnp.int32).reshape(T),
             entity_ids.astype(jnp.int32).reshape(T),
             bboxes.reshape(T * 4))
    features = out.reshape(B, N, F)
    mask = jnp.ones((B, N), dtype=jnp.float32)
    return features, mask


# speedup vs baseline: 6.7150x; 2.5670x over previous
import functools

import jax
import jax.numpy as jnp
from jax import lax
from jax.experimental import pallas as pl
from jax.experimental.pallas import tpu as pltpu
from jax.experimental.pallas import tpu_sc as plsc

_SQRT_HALF = 0.7071067811865476


def _table_proc_body(rt, rwt, rb2, rg2, rbe2, et, ewt, eb2, eg2, ebe2,
                     ro, eo):
    def proc(t_ref, wt_ref, b_ref, g_ref, be_ref, o_ref):
        z = jnp.dot(t_ref[...], wt_ref[...],
                    preferred_element_type=jnp.float32) + b_ref[...]
        m = jnp.mean(z, axis=-1, keepdims=True)
        zc = z - m
        v = jnp.mean(zc * zc, axis=-1, keepdims=True)
        y = zc / jnp.sqrt(v + 1e-5) * g_ref[...] + be_ref[...]
        o_ref[...] = y * 0.5 * (1.0 + lax.erf(y * _SQRT_HALF))

    proc(rt, rwt, rb2, rg2, rbe2, ro)
    proc(et, ewt, eb2, eg2, ebe2, eo)


def _process_tables(region_table, entity_table, rW, rb, r_gamma, r_beta,
                    eW, eb, e_gamma, e_beta):
    VR, D = region_table.shape
    VE, _ = entity_table.shape
    f32 = jnp.float32
    return pl.pallas_call(
        _table_proc_body,
        out_shape=[jax.ShapeDtypeStruct((VR, D), f32),
                   jax.ShapeDtypeStruct((VE, D), f32)],
    )(region_table, rW.T, rb.reshape(1, D), r_gamma.reshape(1, D),
      r_beta.reshape(1, D), entity_table, eW.T, eb.reshape(1, D),
      e_gamma.reshape(1, D), e_beta.reshape(1, D))


def _make_sc_gather(N, B, D, VR, VE, F):
    NC, NS, L = 2, 16, 16
    NW = NC * NS
    NT = N // 8
    BT = B // 128
    NITEMS = NT * BT
    JMAX = (NITEMS + NW - 1) // NW
    SEG = 16
    NSEG = 2 * D // SEG

    mesh = plsc.VectorSubcoreMesh(core_axis_name="c", subcore_axis_name="s")

    @functools.partial(
        pl.kernel, mesh=mesh,
        compiler_params=pltpu.CompilerParams(needs_layout_passes=False),
        out_type=jax.ShapeDtypeStruct((F, N, B), jnp.float32),
        scratch_types=[
            pltpu.VMEM((D * VR,), jnp.float32),
            pltpu.VMEM((D * VE,), jnp.float32),
            pltpu.VMEM((8, 128), jnp.int32),
            pltpu.VMEM((8, 128), jnp.int32),
            pltpu.VMEM((4, 8, 128), jnp.float32),
            pltpu.VMEM((6, 8, 128), jnp.float32),
            pltpu.VMEM((SEG, 8, 128), jnp.float32),
        ],
    )
    def sc_body(prc, pec, ridT, eidT, bbT, out_hbm,
                tbl_r, tbl_e, rid_v, eid_v, bb_v, stgb, stgs):
        wid = lax.axis_index("s") * NC + lax.axis_index("c")
        pltpu.sync_copy(prc, tbl_r)
        pltpu.sync_copy(pec, tbl_e)

        def item_body(j, _):
            item = wid + NW * j

            @pl.when(item < NITEMS)
            def _():
                nt = item // BT
                bt = item - nt * BT
                n0 = nt * 8
                b0 = bt * 128
                pltpu.sync_copy(ridT.at[pl.ds(n0, 8), pl.ds(b0, 128)],
                                rid_v)
                pltpu.sync_copy(eidT.at[pl.ds(n0, 8), pl.ds(b0, 128)],
                                eid_v)
                pltpu.sync_copy(bbT.at[:, pl.ds(n0, 8), pl.ds(b0, 128)],
                                bb_v)

                def bb_row(r, _):
                    for q in range(8):
                        sl = pl.ds(q * L, L)
                        x1 = bb_v[0, r, sl]
                        y1 = bb_v[1, r, sl]
                        x2 = bb_v[2, r, sl]
                        y2 = bb_v[3, r, sl]
                        w = x2 - x1
                        h = y2 - y1
                        stgb[0, r, sl] = x1
                        stgb[1, r, sl] = y1
                        stgb[2, r, sl] = x2
                        stgb[3, r, sl] = y2
                        stgb[4, r, sl] = w * h
                        stgb[5, r, sl] = w / (h + 1e-6)
                    return 0

                lax.fori_loop(0, 8, bb_row, 0)
                pltpu.sync_copy(
                    stgb,
                    out_hbm.at[pl.ds(0, 6), pl.ds(n0, 8), pl.ds(b0, 128)])

                def seg_body(s, ids_v, tbl, V, fbase):
                    c0 = SEG * s

                    def seg_row(r, _):
                        for q in range(8):
                            sl = pl.ds(q * L, L)
                            ids = ids_v[r, sl] + c0 * V
                            for i in range(SEG):
                                v = plsc.load_gather(tbl, [ids + i * V])
                                stgs[i, r, sl] = v
                        return 0

                    lax.fori_loop(0, 8, seg_row, 0)
                    pltpu.sync_copy(
                        stgs,
                        out_hbm.at[pl.ds(fbase + c0, SEG), pl.ds(n0, 8),
                                   pl.ds(b0, 128)])
                    return 0

                lax.fori_loop(
                    0, D // SEG,
                    lambda s, _: seg_body(s, rid_v, tbl_r, VR, 6), 0)
                lax.fori_loop(
                    0, D // SEG,
                    lambda s, _: seg_body(s, eid_v, tbl_e, VE, 6 + D), 0)

            return 0

        lax.fori_loop(0, JMAX, item_body, 0)

    return sc_body


def kernel(bboxes, region_table, entity_table, rW, rb, r_gamma, r_beta,
           eW, eb, e_gamma, e_beta, region_ids, entity_ids):
    B, N, _ = bboxes.shape
    VR, D = region_table.shape
    VE, _ = entity_table.shape
    F = 2 * D + 6

    pr, pe = _process_tables(region_table, entity_table, rW, rb, r_gamma,
                             r_beta, eW, eb, e_gamma, e_beta)
    prc = pr.T.reshape(D * VR)
    pec = pe.T.reshape(D * VE)

    ridT = region_ids.astype(jnp.int32).T
    eidT = entity_ids.astype(jnp.int32).T
    bbT = bboxes.transpose(2, 1, 0)

    sc = _make_sc_gather(N, B, D, VR, VE, F)
    out = sc(prc, pec, ridT, eidT, bbT)
    features = out.transpose(2, 1, 0)
    mask = jnp.ones((B, N), dtype=jnp.float32)
    return features, mask


# speedup vs baseline: 7.4225x; 1.1054x over previous
import functools

import jax
import jax.numpy as jnp
from jax import lax
from jax.experimental import pallas as pl
from jax.experimental.pallas import tpu as pltpu
from jax.experimental.pallas import tpu_sc as plsc

_SQRT_HALF = 0.7071067811865476


def _table_proc_body(rt, rwt, rb2, rg2, rbe2, et, ewt, eb2, eg2, ebe2,
                     ro, eo):
    def proc(t_ref, wt_ref, b_ref, g_ref, be_ref, o_ref):
        z = jnp.dot(t_ref[...], wt_ref[...],
                    preferred_element_type=jnp.float32) + b_ref[...]
        m = jnp.mean(z, axis=-1, keepdims=True)
        zc = z - m
        v = jnp.mean(zc * zc, axis=-1, keepdims=True)
        y = zc / jnp.sqrt(v + 1e-5) * g_ref[...] + be_ref[...]
        o_ref[...] = y * 0.5 * (1.0 + lax.erf(y * _SQRT_HALF))

    proc(rt, rwt, rb2, rg2, rbe2, ro)
    proc(et, ewt, eb2, eg2, ebe2, eo)


def _process_tables(region_table, entity_table, rW, rb, r_gamma, r_beta,
                    eW, eb, e_gamma, e_beta):
    VR, D = region_table.shape
    VE, _ = entity_table.shape
    f32 = jnp.float32
    return pl.pallas_call(
        _table_proc_body,
        out_shape=[jax.ShapeDtypeStruct((VR, D), f32),
                   jax.ShapeDtypeStruct((VE, D), f32)],
    )(region_table, rW.T, rb.reshape(1, D), r_gamma.reshape(1, D),
      r_beta.reshape(1, D), entity_table, eW.T, eb.reshape(1, D),
      e_gamma.reshape(1, D), e_beta.reshape(1, D))


def _make_sc_gather(N, B, D, VR, VE, F):
    NC, NS, L = 2, 16, 16
    NW = NC * NS
    NT = N // 8
    BT = B // 128
    NITEMS = NT * BT
    JMAX = (NITEMS + NW - 1) // NW
    SEG = 32

    mesh = plsc.VectorSubcoreMesh(core_axis_name="c", subcore_axis_name="s")

    @functools.partial(
        pl.kernel, mesh=mesh,
        compiler_params=pltpu.CompilerParams(needs_layout_passes=False),
        out_type=jax.ShapeDtypeStruct((F, N, B), jnp.float32),
        scratch_types=[
            pltpu.VMEM((D * VR,), jnp.float32),
            pltpu.VMEM((D * VE,), jnp.float32),
            pltpu.VMEM((8, 128), jnp.int32),
            pltpu.VMEM((8, 128), jnp.int32),
            pltpu.VMEM((4, 8, 128), jnp.float32),
            pltpu.VMEM((6, 8, 128), jnp.float32),
            pltpu.VMEM((SEG, 8, 128), jnp.float32),
            pltpu.VMEM((SEG, 8, 128), jnp.float32),
            pltpu.SemaphoreType.DMA,
            pltpu.SemaphoreType.DMA,
            pltpu.SemaphoreType.DMA,
        ],
    )
    def sc_body(prc, pec, ridT, eidT, bbT, out_hbm,
                tbl_r, tbl_e, rid_v, eid_v, bb_v, stgb, stgs0, stgs1,
                sem0, sem1, semb):
        wid = lax.axis_index("s") * NC + lax.axis_index("c")
        pltpu.sync_copy(prc, tbl_r)
        pltpu.sync_copy(pec, tbl_e)
        stgs = (stgs0, stgs1)
        sems = (sem0, sem1)

        def item_body(j, _):
            item = wid + NW * j

            @pl.when(item < NITEMS)
            def _():
                nt = item // BT
                bt = item - nt * BT
                n0 = nt * 8
                b0 = bt * 128
                pltpu.sync_copy(ridT.at[pl.ds(n0, 8), pl.ds(b0, 128)],
                                rid_v)
                pltpu.sync_copy(eidT.at[pl.ds(n0, 8), pl.ds(b0, 128)],
                                eid_v)
                pltpu.sync_copy(bbT.at[:, pl.ds(n0, 8), pl.ds(b0, 128)],
                                bb_v)

                def out_slice(fbase, nf):
                    return out_hbm.at[pl.ds(fbase, nf), pl.ds(n0, 8),
                                      pl.ds(b0, 128)]

                @pl.when(j > 0)
                def _():
                    pltpu.make_async_copy(stgb, out_slice(0, 6), semb).wait()

                def bb_row(r, _):
                    for q in range(8):
                        sl = pl.ds(q * L, L)
                        x1 = bb_v[0, r, sl]
                        y1 = bb_v[1, r, sl]
                        x2 = bb_v[2, r, sl]
                        y2 = bb_v[3, r, sl]
                        w = x2 - x1
                        h = y2 - y1
                        stgb[0, r, sl] = x1
                        stgb[1, r, sl] = y1
                        stgb[2, r, sl] = x2
                        stgb[3, r, sl] = y2
                        stgb[4, r, sl] = w * h
                        stgb[5, r, sl] = w / (h + 1e-6)
                    return 0

                lax.fori_loop(0, 8, bb_row, 0)
                pltpu.make_async_copy(stgb, out_slice(0, 6), semb).start()

                def seg_body(k, ids_v, tbl, V, c0, fbase, first):
                    stg = stgs[k]
                    sem = sems[k]
                    if first:
                        @pl.when(j > 0)
                        def _():
                            pltpu.make_async_copy(
                                stg, out_slice(fbase, SEG), sem).wait()
                    else:
                        pltpu.make_async_copy(
                            stg, out_slice(fbase, SEG), sem).wait()

                    def seg_row(r, _):
                        for q in range(8):
                            sl = pl.ds(q * L, L)
                            ids = ids_v[r, sl] + c0 * V
                            for i in range(SEG):
                                v = plsc.load_gather(tbl, [ids + i * V])
                                stg[i, r, sl] = v
                        return 0

                    lax.fori_loop(0, 8, seg_row, 0)
                    pltpu.make_async_copy(
                        stg, out_slice(fbase + c0, SEG), sem).start()

                seg_body(0, rid_v, tbl_r, VR, 0, 6, True)
                seg_body(1, rid_v, tbl_r, VR, SEG, 6, True)
                seg_body(0, eid_v, tbl_e, VE, 0, 6 + D, False)
                seg_body(1, eid_v, tbl_e, VE, SEG, 6 + D, False)

            return 0

        lax.fori_loop(0, JMAX, item_body, 0)
        pltpu.make_async_copy(
            stgb, out_hbm.at[pl.ds(0, 6), pl.ds(0, 8), pl.ds(0, 128)],
            semb).wait()
        for k in (0, 1):
            pltpu.make_async_copy(
                stgs[k],
                out_hbm.at[pl.ds(0, SEG), pl.ds(0, 8), pl.ds(0, 128)],
                sems[k]).wait()

    return sc_body


def kernel(bboxes, region_table, entity_table, rW, rb, r_gamma, r_beta,
           eW, eb, e_gamma, e_beta, region_ids, entity_ids):
    B, N, _ = bboxes.shape
    VR, D = region_table.shape
    VE, _ = entity_table.shape
    F = 2 * D + 6

    pr, pe = _process_tables(region_table, entity_table, rW, rb, r_gamma,
                             r_beta, eW, eb, e_gamma, e_beta)
    prc = pr.T.reshape(D * VR)
    pec = pe.T.reshape(D * VE)

    ridT = region_ids.astype(jnp.int32).T
    eidT = entity_ids.astype(jnp.int32).T
    bbT = bboxes.transpose(2, 1, 0)

    sc = _make_sc_gather(N, B, D, VR, VE, F)
    out = sc(prc, pec, ridT, eidT, bbT)
    features = out.transpose(2, 1, 0)
    mask = jnp.ones((B, N), dtype=jnp.float32)
    return features, mask


# speedup vs baseline: 17.3147x; 2.3327x over previous
import functools

import jax
import jax.numpy as jnp
from jax import lax
from jax.experimental import pallas as pl
from jax.experimental.pallas import tpu as pltpu
from jax.experimental.pallas import tpu_sc as plsc

_SQRT_HALF = 0.7071067811865476


def _table_proc_body(rt, rwt, rb2, rg2, rbe2, et, ewt, eb2, eg2, ebe2,
                     ro, eo):
    def proc(t_ref, wt_ref, b_ref, g_ref, be_ref, o_ref):
        z = jnp.dot(t_ref[...], wt_ref[...],
                    preferred_element_type=jnp.float32) + b_ref[...]
        m = jnp.mean(z, axis=-1, keepdims=True)
        zc = z - m
        v = jnp.mean(zc * zc, axis=-1, keepdims=True)
        y = zc / jnp.sqrt(v + 1e-5) * g_ref[...] + be_ref[...]
        o_ref[...] = y * 0.5 * (1.0 + lax.erf(y * _SQRT_HALF))

    proc(rt, rwt, rb2, rg2, rbe2, ro)
    proc(et, ewt, eb2, eg2, ebe2, eo)


def _process_tables(region_table, entity_table, rW, rb, r_gamma, r_beta,
                    eW, eb, e_gamma, e_beta):
    VR, D = region_table.shape
    VE, _ = entity_table.shape
    f32 = jnp.float32
    return pl.pallas_call(
        _table_proc_body,
        out_shape=[jax.ShapeDtypeStruct((VR, D), f32),
                   jax.ShapeDtypeStruct((VE, D), f32)],
    )(region_table, rW.T, rb.reshape(1, D), r_gamma.reshape(1, D),
      r_beta.reshape(1, D), entity_table, eW.T, eb.reshape(1, D),
      e_gamma.reshape(1, D), e_beta.reshape(1, D))


def _make_sc_gather(N, B, D, VR, VE, F):
    NC, NS, L = 2, 16, 16
    NW = NC * NS
    NT = N // 8
    BT = B // 128
    NITEMS = NT * BT
    JMAX = (NITEMS + NW - 1) // NW
    SEG = 32

    mesh = plsc.VectorSubcoreMesh(core_axis_name="c", subcore_axis_name="s")

    @functools.partial(
        pl.kernel, mesh=mesh,
        compiler_params=pltpu.CompilerParams(needs_layout_passes=False),
        out_type=jax.ShapeDtypeStruct((F, N, B), jnp.float32),
        scratch_types=[
            pltpu.VMEM((D * VR,), jnp.float32),
            pltpu.VMEM((D * VE,), jnp.float32),
            pltpu.VMEM((8, 128), jnp.int32),
            pltpu.VMEM((8, 128), jnp.int32),
            pltpu.VMEM((4, 8, 128), jnp.float32),
            pltpu.VMEM((6, 8, 128), jnp.float32),
            pltpu.VMEM((SEG, 8, 128), jnp.float32),
            pltpu.VMEM((SEG, 8, 128), jnp.float32),
            pltpu.SemaphoreType.DMA,
            pltpu.SemaphoreType.DMA,
            pltpu.SemaphoreType.DMA,
        ],
    )
    def sc_body(prc, pec, ridT, eidT, bbT, out_hbm,
                tbl_r, tbl_e, rid_v, eid_v, bb_v, stgb, stgs0, stgs1,
                sem0, sem1, semb):
        wid = lax.axis_index("s") * NC + lax.axis_index("c")
        pltpu.sync_copy(prc, tbl_r)
        pltpu.sync_copy(pec, tbl_e)
        stgs = (stgs0, stgs1)
        sems = (sem0, sem1)

        def item_body(j, _):
            item = wid + NW * j

            @pl.when(item < NITEMS)
            def _():
                nt = item // BT
                bt = item - nt * BT
                n0 = nt * 8
                b0 = bt * 128
                pltpu.sync_copy(ridT.at[pl.ds(n0, 8), pl.ds(b0, 128)],
                                rid_v)
                pltpu.sync_copy(eidT.at[pl.ds(n0, 8), pl.ds(b0, 128)],
                                eid_v)
                pltpu.sync_copy(bbT.at[:, pl.ds(n0, 8), pl.ds(b0, 128)],
                                bb_v)

                def out_slice(fbase, nf):
                    return out_hbm.at[pl.ds(fbase, nf), pl.ds(n0, 8),
                                      pl.ds(b0, 128)]

                @pl.when(j > 0)
                def _():
                    pltpu.make_async_copy(stgb, out_slice(0, 6), semb).wait()

                def bb_row(r, _):
                    for q in range(8):
                        sl = pl.ds(q * L, L)
                        x1 = bb_v[0, r, sl]
                        y1 = bb_v[1, r, sl]
                        x2 = bb_v[2, r, sl]
                        y2 = bb_v[3, r, sl]
                        w = x2 - x1
                        h = y2 - y1
                        stgb[0, r, sl] = x1
                        stgb[1, r, sl] = y1
                        stgb[2, r, sl] = x2
                        stgb[3, r, sl] = y2
                        stgb[4, r, sl] = w * h
                        stgb[5, r, sl] = w / (h + 1e-6)
                    return 0

                lax.fori_loop(0, 8, bb_row, 0)
                pltpu.make_async_copy(stgb, out_slice(0, 6), semb).start()

                def seg_body(k, ids_v, tbl, V, c0, fbase, first):
                    stg = stgs[k]
                    sem = sems[k]
                    if first:
                        @pl.when(j > 0)
                        def _():
                            pltpu.make_async_copy(
                                stg, out_slice(fbase, SEG), sem).wait()
                    else:
                        pltpu.make_async_copy(
                            stg, out_slice(fbase, SEG), sem).wait()

                    def seg_row(r, _):
                        for q in range(8):
                            sl = pl.ds(q * L, L)
                            ids = ids_v[r, sl] + c0 * V
                            vs = [plsc.load_gather(tbl, [ids + i * V])
                                  for i in range(SEG)]
                            for i in range(SEG):
                                stg[i, r, sl] = vs[i]
                        return 0

                    lax.fori_loop(0, 8, seg_row, 0)
                    pltpu.make_async_copy(
                        stg, out_slice(fbase + c0, SEG), sem).start()

                seg_body(0, rid_v, tbl_r, VR, 0, 6, True)
                seg_body(1, rid_v, tbl_r, VR, SEG, 6, True)
                seg_body(0, eid_v, tbl_e, VE, 0, 6 + D, False)
                seg_body(1, eid_v, tbl_e, VE, SEG, 6 + D, False)

            return 0

        lax.fori_loop(0, JMAX, item_body, 0)
        pltpu.make_async_copy(
            stgb, out_hbm.at[pl.ds(0, 6), pl.ds(0, 8), pl.ds(0, 128)],
            semb).wait()
        for k in (0, 1):
            pltpu.make_async_copy(
                stgs[k],
                out_hbm.at[pl.ds(0, SEG), pl.ds(0, 8), pl.ds(0, 128)],
                sems[k]).wait()

    return sc_body


def kernel(bboxes, region_table, entity_table, rW, rb, r_gamma, r_beta,
           eW, eb, e_gamma, e_beta, region_ids, entity_ids):
    B, N, _ = bboxes.shape
    VR, D = region_table.shape
    VE, _ = entity_table.shape
    F = 2 * D + 6

    pr, pe = _process_tables(region_table, entity_table, rW, rb, r_gamma,
                             r_beta, eW, eb, e_gamma, e_beta)
    prc = pr.T.reshape(D * VR)
    pec = pe.T.reshape(D * VE)

    ridT = region_ids.astype(jnp.int32).T
    eidT = entity_ids.astype(jnp.int32).T
    bbT = bboxes.transpose(2, 1, 0)

    sc = _make_sc_gather(N, B, D, VR, VE, F)
    out = sc(prc, pec, ridT, eidT, bbT)
    features = out.transpose(2, 1, 0)
    mask = jnp.ones((B, N), dtype=jnp.float32)
    return features, mask


# speedup vs baseline: 18.6661x; 1.0781x over previous
import functools

import jax
import jax.numpy as jnp
from jax import lax
from jax.experimental import pallas as pl
from jax.experimental.pallas import tpu as pltpu
from jax.experimental.pallas import tpu_sc as plsc

_SQRT_HALF = 0.7071067811865476


def _table_proc_body(rt, rwt, rb2, rg2, rbe2, et, ewt, eb2, eg2, ebe2,
                     ro, eo):
    def proc(t_ref, wt_ref, b_ref, g_ref, be_ref, o_ref):
        z = jnp.dot(t_ref[...], wt_ref[...],
                    preferred_element_type=jnp.float32) + b_ref[...]
        m = jnp.mean(z, axis=-1, keepdims=True)
        zc = z - m
        v = jnp.mean(zc * zc, axis=-1, keepdims=True)
        y = zc / jnp.sqrt(v + 1e-5) * g_ref[...] + be_ref[...]
        o_ref[...] = y * 0.5 * (1.0 + lax.erf(y * _SQRT_HALF))

    proc(rt, rwt, rb2, rg2, rbe2, ro)
    proc(et, ewt, eb2, eg2, ebe2, eo)


def _process_tables(region_table, entity_table, rW, rb, r_gamma, r_beta,
                    eW, eb, e_gamma, e_beta):
    VR, D = region_table.shape
    VE, _ = entity_table.shape
    f32 = jnp.float32
    return pl.pallas_call(
        _table_proc_body,
        out_shape=[jax.ShapeDtypeStruct((VR, D), f32),
                   jax.ShapeDtypeStruct((VE, D), f32)],
    )(region_table, rW.T, rb.reshape(1, D), r_gamma.reshape(1, D),
      r_beta.reshape(1, D), entity_table, eW.T, eb.reshape(1, D),
      e_gamma.reshape(1, D), e_beta.reshape(1, D))


def _make_sc_gather(N, B, D, VR, VE, F):
    NC, NS, L = 2, 16, 16
    NW = NC * NS
    NT = N // 8
    BT = B // 128
    NITEMS = NT * BT
    JMAX = (NITEMS + NW - 1) // NW
    SEG = 16

    mesh = plsc.VectorSubcoreMesh(core_axis_name="c", subcore_axis_name="s")

    @functools.partial(
        pl.kernel, mesh=mesh,
        compiler_params=pltpu.CompilerParams(needs_layout_passes=False),
        out_type=jax.ShapeDtypeStruct((F, N, B), jnp.float32),
        scratch_types=[
            pltpu.VMEM((D * VR,), jnp.float32),
            pltpu.VMEM((D * VE,), jnp.float32),
            pltpu.VMEM((8, 128), jnp.int32),
            pltpu.VMEM((8, 128), jnp.int32),
            pltpu.VMEM((4, 8, 128), jnp.float32),
            pltpu.VMEM((6, 8, 128), jnp.float32),
            pltpu.VMEM((SEG, 8, 128), jnp.float32),
            pltpu.VMEM((SEG, 8, 128), jnp.float32),
            pltpu.SemaphoreType.DMA,
            pltpu.SemaphoreType.DMA,
            pltpu.SemaphoreType.DMA,
            pltpu.SemaphoreType.DMA,
        ],
    )
    def sc_body(prc, pec, ridT, eidT, bbT, out_hbm,
                tbl_r, tbl_e, rid_v, eid_v, bb_v, stgb, stgs0, stgs1,
                sem0, sem1, semb, semi):
        wid = lax.axis_index("s") * NC + lax.axis_index("c")
        pltpu.sync_copy(prc, tbl_r)
        pltpu.sync_copy(pec, tbl_e)
        stgs = (stgs0, stgs1)
        sems = (sem0, sem1)

        def fire_inputs(jn):
            itemn = wid + NW * jn

            @pl.when(itemn < NITEMS)
            def _():
                ntn = itemn // BT
                btn = itemn - ntn * BT
                n0n = ntn * 8
                b0n = btn * 128
                pltpu.make_async_copy(
                    ridT.at[pl.ds(n0n, 8), pl.ds(b0n, 128)], rid_v,
                    semi).start()
                pltpu.make_async_copy(
                    eidT.at[pl.ds(n0n, 8), pl.ds(b0n, 128)], eid_v,
                    semi).start()
                pltpu.make_async_copy(
                    bbT.at[:, pl.ds(n0n, 8), pl.ds(b0n, 128)], bb_v,
                    semi).start()

        fire_inputs(0)

        def item_body(j, _):
            item = wid + NW * j

            @pl.when(item < NITEMS)
            def _():
                nt = item // BT
                bt = item - nt * BT
                n0 = nt * 8
                b0 = bt * 128
                pltpu.make_async_copy(
                    ridT.at[pl.ds(n0, 8), pl.ds(b0, 128)], rid_v,
                    semi).wait()
                pltpu.make_async_copy(
                    eidT.at[pl.ds(n0, 8), pl.ds(b0, 128)], eid_v,
                    semi).wait()
                pltpu.make_async_copy(
                    bbT.at[:, pl.ds(n0, 8), pl.ds(b0, 128)], bb_v,
                    semi).wait()

                def out_slice(fbase, nf):
                    return out_hbm.at[pl.ds(fbase, nf), pl.ds(n0, 8),
                                      pl.ds(b0, 128)]

                @pl.when(j > 0)
                def _():
                    pltpu.make_async_copy(stgb, out_slice(0, 6), semb).wait()

                def bb_row(r, _):
                    for q in range(8):
                        sl = pl.ds(q * L, L)
                        x1 = bb_v[0, r, sl]
                        y1 = bb_v[1, r, sl]
                        x2 = bb_v[2, r, sl]
                        y2 = bb_v[3, r, sl]
                        w = x2 - x1
                        h = y2 - y1
                        stgb[0, r, sl] = x1
                        stgb[1, r, sl] = y1
                        stgb[2, r, sl] = x2
                        stgb[3, r, sl] = y2
                        stgb[4, r, sl] = w * h
                        stgb[5, r, sl] = w / (h + 1e-6)
                    return 0

                lax.fori_loop(0, 8, bb_row, 0)
                pltpu.make_async_copy(stgb, out_slice(0, 6), semb).start()

                def seg_body(k, ids_v, tbl, V, c0, fbase, first):
                    stg = stgs[k]
                    sem = sems[k]
                    if first:
                        @pl.when(j > 0)
                        def _():
                            pltpu.make_async_copy(
                                stg, out_slice(fbase, SEG), sem).wait()
                    else:
                        pltpu.make_async_copy(
                            stg, out_slice(fbase, SEG), sem).wait()

                    def seg_row(r, _):
                        vs_prev = sl_prev = None
                        for q in range(8):
                            sl = pl.ds(q * L, L)
                            ids = ids_v[r, sl] + c0 * V
                            vs = [plsc.load_gather(tbl, [ids + i * V])
                                  for i in range(SEG)]
                            if vs_prev is not None:
                                for i in range(SEG):
                                    stg[i, r, sl_prev] = vs_prev[i]
                            vs_prev, sl_prev = vs, sl
                        for i in range(SEG):
                            stg[i, r, sl_prev] = vs_prev[i]
                        return 0

                    lax.fori_loop(0, 8, seg_row, 0)
                    pltpu.make_async_copy(
                        stg, out_slice(fbase + c0, SEG), sem).start()

                for s in range(D // SEG):
                    seg_body(s % 2, rid_v, tbl_r, VR, SEG * s, 6,
                             s < 2)
                for s in range(D // SEG):
                    seg_body(s % 2, eid_v, tbl_e, VE, SEG * s, 6 + D,
                             False)

                fire_inputs(j + 1)

            return 0

        lax.fori_loop(0, JMAX, item_body, 0)
        pltpu.make_async_copy(
            stgb, out_hbm.at[pl.ds(0, 6), pl.ds(0, 8), pl.ds(0, 128)],
            semb).wait()
        for k in (0, 1):
            pltpu.make_async_copy(
                stgs[k],
                out_hbm.at[pl.ds(0, SEG), pl.ds(0, 8), pl.ds(0, 128)],
                sems[k]).wait()

    return sc_body


def kernel(bboxes, region_table, entity_table, rW, rb, r_gamma, r_beta,
           eW, eb, e_gamma, e_beta, region_ids, entity_ids):
    B, N, _ = bboxes.shape
    VR, D = region_table.shape
    VE, _ = entity_table.shape
    F = 2 * D + 6

    pr, pe = _process_tables(region_table, entity_table, rW, rb, r_gamma,
                             r_beta, eW, eb, e_gamma, e_beta)
    prc = pr.T.reshape(D * VR)
    pec = pe.T.reshape(D * VE)

    ridT = region_ids.astype(jnp.int32).T
    eidT = entity_ids.astype(jnp.int32).T
    bbT = bboxes.transpose(2, 1, 0)

    sc = _make_sc_gather(N, B, D, VR, VE, F)
    out = sc(prc, pec, ridT, eidT, bbT)
    features = out.transpose(2, 1, 0)
    mask = jnp.ones((B, N), dtype=jnp.float32)
    return features, mask


# speedup vs baseline: 20.3557x; 1.0905x over previous
import functools

import jax
import jax.numpy as jnp
from jax import lax
from jax.experimental import pallas as pl
from jax.experimental.pallas import tpu as pltpu
from jax.experimental.pallas import tpu_sc as plsc

_SQRT_HALF = 0.7071067811865476


def _table_proc_body(rt, rwt, rb2, rg2, rbe2, et, ewt, eb2, eg2, ebe2,
                     ro, eo):
    def proc(t_ref, wt_ref, b_ref, g_ref, be_ref, o_ref):
        z = jnp.dot(t_ref[...], wt_ref[...],
                    preferred_element_type=jnp.float32) + b_ref[...]
        m = jnp.mean(z, axis=-1, keepdims=True)
        zc = z - m
        v = jnp.mean(zc * zc, axis=-1, keepdims=True)
        y = zc / jnp.sqrt(v + 1e-5) * g_ref[...] + be_ref[...]
        o_ref[...] = y * 0.5 * (1.0 + lax.erf(y * _SQRT_HALF))

    proc(rt, rwt, rb2, rg2, rbe2, ro)
    proc(et, ewt, eb2, eg2, ebe2, eo)


def _process_tables(region_table, entity_table, rW, rb, r_gamma, r_beta,
                    eW, eb, e_gamma, e_beta):
    VR, D = region_table.shape
    VE, _ = entity_table.shape
    f32 = jnp.float32
    return pl.pallas_call(
        _table_proc_body,
        out_shape=[jax.ShapeDtypeStruct((VR, D), f32),
                   jax.ShapeDtypeStruct((VE, D), f32)],
    )(region_table, rW.T, rb.reshape(1, D), r_gamma.reshape(1, D),
      r_beta.reshape(1, D), entity_table, eW.T, eb.reshape(1, D),
      e_gamma.reshape(1, D), e_beta.reshape(1, D))


def _make_sc_gather(N, B, D, VR, VE, F):
    NC, NS, L = 2, 16, 16
    NW = NC * NS
    NT = N // 8
    BT = B // 128
    NITEMS = NT * BT
    JMAX = (NITEMS + NW - 1) // NW
    SEG = 16

    mesh = plsc.VectorSubcoreMesh(core_axis_name="c", subcore_axis_name="s")

    @functools.partial(
        pl.kernel, mesh=mesh,
        compiler_params=pltpu.CompilerParams(needs_layout_passes=False),
        out_type=jax.ShapeDtypeStruct((F, N, B), jnp.float32),
        scratch_types=[
            pltpu.VMEM((D * VR,), jnp.float32),
            pltpu.VMEM((D * VE,), jnp.float32),
            pltpu.VMEM((8, 128), jnp.int32),
            pltpu.VMEM((8, 128), jnp.int32),
            pltpu.VMEM((4, 8, 128), jnp.float32),
            pltpu.VMEM((6, 8, 128), jnp.float32),
            pltpu.VMEM((SEG, 8, 128), jnp.float32),
            pltpu.VMEM((SEG, 8, 128), jnp.float32),
            pltpu.SemaphoreType.DMA,
            pltpu.SemaphoreType.DMA,
            pltpu.SemaphoreType.DMA,
            pltpu.SemaphoreType.DMA,
        ],
    )
    def sc_body(prc, pec, ridT, eidT, bbT, out_hbm,
                tbl_r, tbl_e, rid_v, eid_v, bb_v, stgb, stgs0, stgs1,
                sem0, sem1, semb, semi):
        wid = lax.axis_index("s") * NC + lax.axis_index("c")
        pltpu.sync_copy(prc, tbl_r)
        pltpu.sync_copy(pec, tbl_e)
        stgs = (stgs0, stgs1)
        sems = (sem0, sem1)

        def fire_inputs(jn):
            itemn = wid + NW * jn

            @pl.when(itemn < NITEMS)
            def _():
                ntn = itemn // BT
                btn = itemn - ntn * BT
                n0n = ntn * 8
                b0n = btn * 128
                pltpu.make_async_copy(
                    ridT.at[pl.ds(n0n, 8), pl.ds(b0n, 128)], rid_v,
                    semi).start()
                pltpu.make_async_copy(
                    eidT.at[pl.ds(n0n, 8), pl.ds(b0n, 128)], eid_v,
                    semi).start()
                pltpu.make_async_copy(
                    bbT.at[:, pl.ds(n0n, 8), pl.ds(b0n, 128)], bb_v,
                    semi).start()

        fire_inputs(0)

        def item_body(j, _):
            item = wid + NW * j

            @pl.when(item < NITEMS)
            def _():
                nt = item // BT
                bt = item - nt * BT
                n0 = nt * 8
                b0 = bt * 128
                pltpu.make_async_copy(
                    ridT.at[pl.ds(n0, 8), pl.ds(b0, 128)], rid_v,
                    semi).wait()
                pltpu.make_async_copy(
                    eidT.at[pl.ds(n0, 8), pl.ds(b0, 128)], eid_v,
                    semi).wait()
                pltpu.make_async_copy(
                    bbT.at[:, pl.ds(n0, 8), pl.ds(b0, 128)], bb_v,
                    semi).wait()

                def out_slice(fbase, nf):
                    return out_hbm.at[pl.ds(fbase, nf), pl.ds(n0, 8),
                                      pl.ds(b0, 128)]

                @pl.when(j > 0)
                def _():
                    pltpu.make_async_copy(stgb, out_slice(0, 6), semb).wait()

                @plsc.parallel_loop(0, 8)
                def bb_row(r):
                    for q in range(8):
                        sl = pl.ds(q * L, L)
                        x1 = bb_v[0, r, sl]
                        y1 = bb_v[1, r, sl]
                        x2 = bb_v[2, r, sl]
                        y2 = bb_v[3, r, sl]
                        w = x2 - x1
                        h = y2 - y1
                        stgb[0, r, sl] = x1
                        stgb[1, r, sl] = y1
                        stgb[2, r, sl] = x2
                        stgb[3, r, sl] = y2
                        stgb[4, r, sl] = w * h
                        stgb[5, r, sl] = w / (h + 1e-6)
                pltpu.make_async_copy(stgb, out_slice(0, 6), semb).start()

                def seg_body(k, ids_v, tbl, V, c0, fbase, first):
                    stg = stgs[k]
                    sem = sems[k]
                    if first:
                        @pl.when(j > 0)
                        def _():
                            pltpu.make_async_copy(
                                stg, out_slice(fbase, SEG), sem).wait()
                    else:
                        pltpu.make_async_copy(
                            stg, out_slice(fbase, SEG), sem).wait()

                    @plsc.parallel_loop(0, 8, unroll=2)
                    def seg_row(r):
                        vs_prev = sl_prev = None
                        for q in range(8):
                            sl = pl.ds(q * L, L)
                            ids = ids_v[r, sl] + c0 * V
                            vs = [plsc.load_gather(tbl, [ids + i * V])
                                  for i in range(SEG)]
                            if vs_prev is not None:
                                for i in range(SEG):
                                    stg[i, r, sl_prev] = vs_prev[i]
                            vs_prev, sl_prev = vs, sl
                        for i in range(SEG):
                            stg[i, r, sl_prev] = vs_prev[i]
                    pltpu.make_async_copy(
                        stg, out_slice(fbase + c0, SEG), sem).start()

                for s in range(D // SEG):
                    seg_body(s % 2, rid_v, tbl_r, VR, SEG * s, 6,
                             s < 2)
                for s in range(D // SEG):
                    seg_body(s % 2, eid_v, tbl_e, VE, SEG * s, 6 + D,
                             False)

                fire_inputs(j + 1)

            return 0

        lax.fori_loop(0, JMAX, item_body, 0)
        pltpu.make_async_copy(
            stgb, out_hbm.at[pl.ds(0, 6), pl.ds(0, 8), pl.ds(0, 128)],
            semb).wait()
        for k in (0, 1):
            pltpu.make_async_copy(
                stgs[k],
                out_hbm.at[pl.ds(0, SEG), pl.ds(0, 8), pl.ds(0, 128)],
                sems[k]).wait()

    return sc_body


def kernel(bboxes, region_table, entity_table, rW, rb, r_gamma, r_beta,
           eW, eb, e_gamma, e_beta, region_ids, entity_ids):
    B, N, _ = bboxes.shape
    VR, D = region_table.shape
    VE, _ = entity_table.shape
    F = 2 * D + 6

    pr, pe = _process_tables(region_table, entity_table, rW, rb, r_gamma,
                             r_beta, eW, eb, e_gamma, e_beta)
    prc = pr.T.reshape(D * VR)
    pec = pe.T.reshape(D * VE)

    ridT = region_ids.astype(jnp.int32).T
    eidT = entity_ids.astype(jnp.int32).T
    bbT = bboxes.transpose(2, 1, 0)

    sc = _make_sc_gather(N, B, D, VR, VE, F)
    out = sc(prc, pec, ridT, eidT, bbT)
    features = out.transpose(2, 1, 0)
    mask = jnp.ones((B, N), dtype=jnp.float32)
    return features, mask
